# baseline (speedup 1.0000x reference)
import functools

import jax
import jax.numpy as jnp
from jax import lax
from jax.experimental import pallas as pl
from jax.experimental.pallas import tpu as pltpu
from jax.experimental.pallas import tpu_sc as plsc

N = 100000
N_PAD = 100096
NODES_PER_TILE = N_PAD // 16
WB_CHUNK = 368
WB_COPIES = NODES_PER_TILE // WB_CHUNK

SUB = 64
CHR = 8
CH = SUB * CHR
NCHUNKS = 6250


def _linear_relu_body(x_ref, w_ref, b_ref, o_ref):
    acc = jnp.dot(w_ref[...], x_ref[...], preferred_element_type=jnp.float32)
    o_ref[...] = jnp.maximum(acc + b_ref[...], 0.0)


def _linear_relu_t(emb_t, w, b_col):
    cols = emb_t.shape[1]
    blk = 2560
    grid = cols // blk
    return pl.pallas_call(
        _linear_relu_body,
        grid=(grid,),
        in_specs=[
            pl.BlockSpec((16, blk), lambda i: (0, i)),
            pl.BlockSpec((16, 16), lambda i: (0, 0)),
            pl.BlockSpec((16, 1), lambda i: (0, 0)),
        ],
        out_specs=pl.BlockSpec((16, blk), lambda i: (0, i)),
        out_shape=jax.ShapeDtypeStruct((16, cols), jnp.float32),
    )(emb_t, w, b_col)


def _scatter_body(ht_hbm, idx_hbm, sums_hbm, cnts_hbm,
                  buft_v, rows_v, idx_v, ones_v, z1_v, sem_s, sem_c,
                  acc_sum, acc_cnt):
    num_edges = ht_hbm.shape[1]
    idx_rows_per_sec = num_edges // SUB
    cid = lax.axis_index("c")
    sid = lax.axis_index("s")
    nch = (NCHUNKS + 15 - sid) // 16

    for i in range(SUB // 16):
        ones_v[pl.ds(i * 16, 16)] = jnp.full((16,), 1.0, jnp.float32)

    def zr(i, _):
        rows_v[i] = jnp.zeros((16,), jnp.float32)
        return 0
    lax.fori_loop(0, WB_CHUNK, zr, 0)

    def zc(i, _):
        z1_v[pl.ds(i * 16, 16)] = jnp.zeros((16,), jnp.float32)
        return 0
    lax.fori_loop(0, WB_CHUNK // 16, zc, 0)

    node0 = sid * NODES_PER_TILE
    for k in range(WB_COPIES):
        off = pl.multiple_of(node0 + k * WB_CHUNK, 8)
        pltpu.sync_copy(rows_v.at[pl.ds(0, WB_CHUNK)],
                        acc_sum.at[pl.ds(off, WB_CHUNK)])
        pltpu.sync_copy(z1_v, acc_cnt.at[pl.ds(off, WB_CHUNK)])
    plsc.subcore_barrier()

    lane_iota = jnp.arange(16, dtype=jnp.int32)

    def chunk(i, _):
        c = 16 * i + sid
        irow = pl.multiple_of(cid * idx_rows_per_sec + c * CHR, 8)
        ecol = pl.multiple_of(c * CH, 8)
        pltpu.sync_copy(idx_hbm.at[pl.ds(irow, CHR)], idx_v)
        pltpu.sync_copy(ht_hbm.at[:, pl.ds(ecol, CH)], buft_v)

        def tp(g, _):
            for k in range(8):
                e = g * 8 + k
                col = plsc.load_gather(
                    buft_v, [lane_iota, jnp.full((16,), e, jnp.int32)])
                rows_v[e] = col
            return 0
        lax.fori_loop(0, CH // 8, tp, 0)
        cps = []
        for j in range(CHR):
            cps.append(pltpu.async_copy(rows_v.at[pl.ds(j * SUB, SUB)],
                                        acc_sum.at[idx_v.at[j]], sem_s,
                                        add=True))
            cps.append(pltpu.async_copy(ones_v, acc_cnt.at[idx_v.at[j]],
                                        sem_c, add=True))
        for cp in cps:
            cp.wait()
        return 0
    lax.fori_loop(0, nch, chunk, 0)
    plsc.subcore_barrier()

    out0 = cid * N_PAD + node0
    for k in range(WB_COPIES):
        off = pl.multiple_of(node0 + k * WB_CHUNK, 8)
        off_o = pl.multiple_of(out0 + k * WB_CHUNK, 8)
        pltpu.sync_copy(acc_sum.at[pl.ds(off, WB_CHUNK)],
                        rows_v.at[pl.ds(0, WB_CHUNK)])
        pltpu.sync_copy(rows_v.at[pl.ds(0, WB_CHUNK)],
                        sums_hbm.at[pl.ds(off_o, WB_CHUNK)])
        pltpu.sync_copy(acc_cnt.at[pl.ds(off, WB_CHUNK)], z1_v)
        pltpu.sync_copy(z1_v, cnts_hbm.at[pl.ds(off_o, WB_CHUNK)])


def _sc_scatter(h, idx2d):
    mesh = plsc.VectorSubcoreMesh(core_axis_name="c", subcore_axis_name="s",
                                  num_cores=2, num_subcores=16)
    f = pl.kernel(
        _scatter_body,
        out_type=[
            jax.ShapeDtypeStruct((2 * N_PAD, 16), jnp.float32),
            jax.ShapeDtypeStruct((2 * N_PAD,), jnp.float32),
        ],
        mesh=mesh,
        scratch_types=[
            pltpu.VMEM((16, CH), jnp.float32),
            pltpu.VMEM((CH, 16), jnp.float32),
            pltpu.VMEM((CHR, SUB), jnp.int32),
            pltpu.VMEM((SUB,), jnp.float32),
            pltpu.VMEM((WB_CHUNK,), jnp.float32),
            pltpu.SemaphoreType.DMA,
            pltpu.SemaphoreType.DMA,
            pltpu.VMEM_SHARED((N_PAD, 16), jnp.float32),
            pltpu.VMEM_SHARED((N_PAD,), jnp.float32),
        ],
        compiler_params=pltpu.CompilerParams(use_tc_tiling_on_sc=False,
                                             needs_layout_passes=False),
    )
    return f(h, idx2d)


def _combine_body(s0_ref, s1_ref, c0_ref, c1_ref, o_ref):
    m0 = s0_ref[...] / jnp.maximum(c0_ref[...], 1.0)
    m1 = s1_ref[...] / jnp.maximum(c1_ref[...], 1.0)
    o_ref[...] = 0.5 * (m0 + m1)


def _combine(s0, s1, c0, c1):
    blk = 1088
    grid = N_PAD // blk
    return pl.pallas_call(
        _combine_body,
        grid=(grid,),
        in_specs=[
            pl.BlockSpec((blk, 16), lambda i: (i, 0)),
            pl.BlockSpec((blk, 16), lambda i: (i, 0)),
            pl.BlockSpec((blk, 1), lambda i: (i, 0)),
            pl.BlockSpec((blk, 1), lambda i: (i, 0)),
        ],
        out_specs=pl.BlockSpec((blk, 16), lambda i: (i, 0)),
        out_shape=jax.ShapeDtypeStruct((N_PAD, 16), jnp.float32),
    )(s0, s1, c0, c1)


def kernel(edge_emb, edge_index, num_nodes, W, b):
    num_edges = edge_emb.shape[0]
    ht = _linear_relu_t(edge_emb.T, W, b.reshape(16, 1))

    idx2d = edge_index.reshape(2 * num_edges // SUB, SUB)
    sums, cnts = _sc_scatter(ht, idx2d)

    s0, s1 = sums[:N_PAD], sums[N_PAD:]
    c0 = cnts[:N_PAD].reshape(N_PAD, 1)
    c1 = cnts[N_PAD:].reshape(N_PAD, 1)
    out = _combine(s0, s1, c0, c1)
    return out[:N]


# speedup vs baseline: 1.6028x; 1.6028x over previous
import functools

import jax
import jax.numpy as jnp
from jax import lax
from jax.experimental import pallas as pl
from jax.experimental.pallas import tpu as pltpu
from jax.experimental.pallas import tpu_sc as plsc

N = 100000
N_PAD = 100096
NODES_PER_TILE = N_PAD // 16
WB_CHUNK = 368
WB_COPIES = NODES_PER_TILE // WB_CHUNK

SUB = 64
CHR = 8
CH = SUB * CHR
NCHUNKS = 6250


def _linear_relu_body(x_ref, w_ref, b_ref, o_ref):
    acc = jnp.dot(w_ref[...], x_ref[...], preferred_element_type=jnp.float32)
    o_ref[...] = jnp.maximum(acc + b_ref[...], 0.0)


def _linear_relu_t(emb_t, w, b_col):
    cols = emb_t.shape[1]
    blk = 2560
    grid = cols // blk
    return pl.pallas_call(
        _linear_relu_body,
        grid=(grid,),
        in_specs=[
            pl.BlockSpec((16, blk), lambda i: (0, i)),
            pl.BlockSpec((16, 16), lambda i: (0, 0)),
            pl.BlockSpec((16, 1), lambda i: (0, 0)),
        ],
        out_specs=pl.BlockSpec((16, blk), lambda i: (0, i)),
        out_shape=jax.ShapeDtypeStruct((16, cols), jnp.float32),
    )(emb_t, w, b_col)


def _scatter_body(ht_hbm, idx_hbm, sums_hbm, cnts_hbm,
                  buft_v, rows_v, idx_v, ones_v, z1_v, sem_s, sem_c,
                  acc_sum, acc_cnt):
    num_edges = ht_hbm.shape[1]
    idx_rows_per_sec = num_edges // SUB
    cid = lax.axis_index("c")
    sid = lax.axis_index("s")
    nch = (NCHUNKS + 15 - sid) // 16

    for i in range(SUB // 16):
        ones_v[pl.ds(i * 16, 16)] = jnp.full((16,), 1.0, jnp.float32)

    def zr(i, _):
        rows_v[i] = jnp.zeros((16,), jnp.float32)
        return 0
    lax.fori_loop(0, WB_CHUNK, zr, 0)

    def zc(i, _):
        z1_v[pl.ds(i * 16, 16)] = jnp.zeros((16,), jnp.float32)
        return 0
    lax.fori_loop(0, WB_CHUNK // 16, zc, 0)

    node0 = sid * NODES_PER_TILE
    for k in range(WB_COPIES):
        off = pl.multiple_of(node0 + k * WB_CHUNK, 8)
        pltpu.sync_copy(rows_v.at[pl.ds(0, WB_CHUNK)],
                        acc_sum.at[pl.ds(off, WB_CHUNK)])
        pltpu.sync_copy(z1_v, acc_cnt.at[pl.ds(off, WB_CHUNK)])
    plsc.subcore_barrier()

    lane_iota = jnp.arange(16, dtype=jnp.int32)

    def chunk(i, _):
        c = 16 * i + sid
        irow = pl.multiple_of(cid * idx_rows_per_sec + c * CHR, 8)
        ecol = pl.multiple_of(c * CH, 8)
        pltpu.sync_copy(idx_hbm.at[pl.ds(irow, CHR)], idx_v)
        pltpu.sync_copy(ht_hbm.at[:, pl.ds(ecol, CH)], buft_v)

        def tp(g, _):
            rows_idx = g * 16 + lane_iota
            for ch in range(16):
                v = buft_v[ch, pl.ds(g * 16, 16)]
                plsc.store_scatter(
                    rows_v, [rows_idx, jnp.full((16,), ch, jnp.int32)], v)
            return 0
        lax.fori_loop(0, CH // 16, tp, 0)
        cps = []
        for j in range(CHR):
            cps.append(pltpu.async_copy(rows_v.at[pl.ds(j * SUB, SUB)],
                                        acc_sum.at[idx_v.at[j]], sem_s,
                                        add=True))
            cps.append(pltpu.async_copy(ones_v, acc_cnt.at[idx_v.at[j]],
                                        sem_c, add=True))
        for cp in cps:
            cp.wait()
        return 0
    lax.fori_loop(0, nch, chunk, 0)
    plsc.subcore_barrier()

    out0 = cid * N_PAD + node0
    for k in range(WB_COPIES):
        off = pl.multiple_of(node0 + k * WB_CHUNK, 8)
        off_o = pl.multiple_of(out0 + k * WB_CHUNK, 8)
        pltpu.sync_copy(acc_sum.at[pl.ds(off, WB_CHUNK)],
                        rows_v.at[pl.ds(0, WB_CHUNK)])
        pltpu.sync_copy(rows_v.at[pl.ds(0, WB_CHUNK)],
                        sums_hbm.at[pl.ds(off_o, WB_CHUNK)])
        pltpu.sync_copy(acc_cnt.at[pl.ds(off, WB_CHUNK)], z1_v)
        pltpu.sync_copy(z1_v, cnts_hbm.at[pl.ds(off_o, WB_CHUNK)])


def _sc_scatter(h, idx2d):
    mesh = plsc.VectorSubcoreMesh(core_axis_name="c", subcore_axis_name="s",
                                  num_cores=2, num_subcores=16)
    f = pl.kernel(
        _scatter_body,
        out_type=[
            jax.ShapeDtypeStruct((2 * N_PAD, 16), jnp.float32),
            jax.ShapeDtypeStruct((2 * N_PAD,), jnp.float32),
        ],
        mesh=mesh,
        scratch_types=[
            pltpu.VMEM((16, CH), jnp.float32),
            pltpu.VMEM((CH, 16), jnp.float32),
            pltpu.VMEM((CHR, SUB), jnp.int32),
            pltpu.VMEM((SUB,), jnp.float32),
            pltpu.VMEM((WB_CHUNK,), jnp.float32),
            pltpu.SemaphoreType.DMA,
            pltpu.SemaphoreType.DMA,
            pltpu.VMEM_SHARED((N_PAD, 16), jnp.float32),
            pltpu.VMEM_SHARED((N_PAD,), jnp.float32),
        ],
        compiler_params=pltpu.CompilerParams(use_tc_tiling_on_sc=False,
                                             needs_layout_passes=False),
    )
    return f(h, idx2d)


def _combine_body(s0_ref, s1_ref, c0_ref, c1_ref, o_ref):
    m0 = s0_ref[...] / jnp.maximum(c0_ref[...], 1.0)
    m1 = s1_ref[...] / jnp.maximum(c1_ref[...], 1.0)
    o_ref[...] = 0.5 * (m0 + m1)


def _combine(s0, s1, c0, c1):
    blk = 1088
    grid = N_PAD // blk
    return pl.pallas_call(
        _combine_body,
        grid=(grid,),
        in_specs=[
            pl.BlockSpec((blk, 16), lambda i: (i, 0)),
            pl.BlockSpec((blk, 16), lambda i: (i, 0)),
            pl.BlockSpec((blk, 1), lambda i: (i, 0)),
            pl.BlockSpec((blk, 1), lambda i: (i, 0)),
        ],
        out_specs=pl.BlockSpec((blk, 16), lambda i: (i, 0)),
        out_shape=jax.ShapeDtypeStruct((N_PAD, 16), jnp.float32),
    )(s0, s1, c0, c1)


def kernel(edge_emb, edge_index, num_nodes, W, b):
    num_edges = edge_emb.shape[0]
    ht = _linear_relu_t(edge_emb.T, W, b.reshape(16, 1))

    idx2d = edge_index.reshape(2 * num_edges // SUB, SUB)
    sums, cnts = _sc_scatter(ht, idx2d)

    s0, s1 = sums[:N_PAD], sums[N_PAD:]
    c0 = cnts[:N_PAD].reshape(N_PAD, 1)
    c1 = cnts[N_PAD:].reshape(N_PAD, 1)
    out = _combine(s0, s1, c0, c1)
    return out[:N]


# speedup vs baseline: 1.8228x; 1.1373x over previous
import jax
import jax.numpy as jnp
from jax import lax
from jax.experimental import pallas as pl
from jax.experimental.pallas import tpu as pltpu
from jax.experimental.pallas import tpu_sc as plsc

N = 100000
N_PAD = 100096
NODES_PER_TILE = N_PAD // 16
WB_CHUNK = 368
WB_COPIES = NODES_PER_TILE // WB_CHUNK

SUB = 128
CHR = 8
CH = SUB * CHR
NCHUNKS = 3125


def _linear_relu_body(x_ref, w_ref, b_ref, o_ref):
    acc = jnp.dot(x_ref[...], w_ref[...], preferred_element_type=jnp.float32)
    o_ref[...] = jnp.maximum(acc + b_ref[...], 0.0)


def _linear_relu(emb_packed, w_block, b_tile):
    rows = emb_packed.shape[0]
    blk = 1000
    grid = rows // blk
    return pl.pallas_call(
        _linear_relu_body,
        grid=(grid,),
        in_specs=[
            pl.BlockSpec((blk, 128), lambda i: (i, 0)),
            pl.BlockSpec((128, 128), lambda i: (0, 0)),
            pl.BlockSpec((1, 128), lambda i: (0, 0)),
        ],
        out_specs=pl.BlockSpec((blk, 128), lambda i: (i, 0)),
        out_shape=jax.ShapeDtypeStruct((rows, 128), jnp.float32),
    )(emb_packed, w_block, b_tile)


def _scatter_body(h_hbm, idx_hbm, sums_hbm, cnts_hbm,
                  rows_v, idx_v, ones_v, z1_v, sem_s, sem_c,
                  acc_sum, acc_cnt):
    num_edges = h_hbm.shape[0]
    idx_rows_per_sec = num_edges // SUB
    cid = lax.axis_index("c")
    sid = lax.axis_index("s")
    nch = (NCHUNKS + 15 - sid) // 16

    for i in range(SUB // 16):
        ones_v[pl.ds(i * 16, 16)] = jnp.full((16,), 1.0, jnp.float32)

    def zr(i, _):
        rows_v[i] = jnp.zeros((16,), jnp.float32)
        return 0
    lax.fori_loop(0, WB_CHUNK, zr, 0)

    def zc(i, _):
        z1_v[pl.ds(i * 16, 16)] = jnp.zeros((16,), jnp.float32)
        return 0
    lax.fori_loop(0, WB_CHUNK // 16, zc, 0)

    node0 = sid * NODES_PER_TILE
    for k in range(WB_COPIES):
        off = pl.multiple_of(node0 + k * WB_CHUNK, 8)
        pltpu.sync_copy(rows_v.at[pl.ds(0, WB_CHUNK)],
                        acc_sum.at[pl.ds(off, WB_CHUNK)])
        pltpu.sync_copy(z1_v, acc_cnt.at[pl.ds(off, WB_CHUNK)])
    plsc.subcore_barrier()

    def chunk(i, _):
        c = 16 * i + sid
        irow = pl.multiple_of(cid * idx_rows_per_sec + c * CHR, 8)
        erow = pl.multiple_of(c * CH, 8)
        pltpu.sync_copy(idx_hbm.at[pl.ds(irow, CHR)], idx_v)
        pltpu.sync_copy(h_hbm.at[pl.ds(erow, CH)], rows_v)
        cps = []
        for j in range(CHR):
            cps.append(pltpu.async_copy(rows_v.at[pl.ds(j * SUB, SUB)],
                                        acc_sum.at[idx_v.at[j]], sem_s,
                                        add=True))
            cps.append(pltpu.async_copy(ones_v, acc_cnt.at[idx_v.at[j]],
                                        sem_c, add=True))
        for cp in cps:
            cp.wait()
        return 0
    lax.fori_loop(0, nch, chunk, 0)
    plsc.subcore_barrier()

    out0 = cid * N_PAD + node0
    for k in range(WB_COPIES):
        off = pl.multiple_of(node0 + k * WB_CHUNK, 8)
        off_o = pl.multiple_of(out0 + k * WB_CHUNK, 8)
        pltpu.sync_copy(acc_sum.at[pl.ds(off, WB_CHUNK)],
                        rows_v.at[pl.ds(0, WB_CHUNK)])
        pltpu.sync_copy(rows_v.at[pl.ds(0, WB_CHUNK)],
                        sums_hbm.at[pl.ds(off_o, WB_CHUNK)])
        pltpu.sync_copy(acc_cnt.at[pl.ds(off, WB_CHUNK)], z1_v)
        pltpu.sync_copy(z1_v, cnts_hbm.at[pl.ds(off_o, WB_CHUNK)])


def _sc_scatter(h, idx2d):
    mesh = plsc.VectorSubcoreMesh(core_axis_name="c", subcore_axis_name="s",
                                  num_cores=2, num_subcores=16)
    f = pl.kernel(
        _scatter_body,
        out_type=[
            jax.ShapeDtypeStruct((2 * N_PAD, 16), jnp.float32),
            jax.ShapeDtypeStruct((2 * N_PAD,), jnp.float32),
        ],
        mesh=mesh,
        scratch_types=[
            pltpu.VMEM((CH, 16), jnp.float32),
            pltpu.VMEM((CHR, SUB), jnp.int32),
            pltpu.VMEM((SUB,), jnp.float32),
            pltpu.VMEM((WB_CHUNK,), jnp.float32),
            pltpu.SemaphoreType.DMA,
            pltpu.SemaphoreType.DMA,
            pltpu.VMEM_SHARED((N_PAD, 16), jnp.float32),
            pltpu.VMEM_SHARED((N_PAD,), jnp.float32),
        ],
        compiler_params=pltpu.CompilerParams(use_tc_tiling_on_sc=False),
    )
    return f(h, idx2d)


def _combine_body(s0_ref, s1_ref, c0_ref, c1_ref, o_ref):
    m0 = s0_ref[...] / jnp.maximum(c0_ref[...], 1.0)
    m1 = s1_ref[...] / jnp.maximum(c1_ref[...], 1.0)
    o_ref[...] = 0.5 * (m0 + m1)


def _combine(s0, s1, c0, c1):
    blk = 1088
    grid = N_PAD // blk
    return pl.pallas_call(
        _combine_body,
        grid=(grid,),
        in_specs=[
            pl.BlockSpec((blk, 16), lambda i: (i, 0)),
            pl.BlockSpec((blk, 16), lambda i: (i, 0)),
            pl.BlockSpec((blk, 1), lambda i: (i, 0)),
            pl.BlockSpec((blk, 1), lambda i: (i, 0)),
        ],
        out_specs=pl.BlockSpec((blk, 16), lambda i: (i, 0)),
        out_shape=jax.ShapeDtypeStruct((N_PAD, 16), jnp.float32),
    )(s0, s1, c0, c1)


def kernel(edge_emb, edge_index, num_nodes, W, b):
    num_edges = edge_emb.shape[0]
    emb_packed = edge_emb.reshape(num_edges // 8, 128)
    w_block = jnp.kron(jnp.eye(8, dtype=jnp.float32), W.T)
    b_tile = jnp.tile(b, 8).reshape(1, 128)
    h = _linear_relu(emb_packed, w_block, b_tile).reshape(num_edges, 16)

    idx2d = edge_index.reshape(2 * num_edges // SUB, SUB)
    sums, cnts = _sc_scatter(h, idx2d)

    s0, s1 = sums[:N_PAD], sums[N_PAD:]
    c0 = cnts[:N_PAD].reshape(N_PAD, 1)
    c1 = cnts[N_PAD:].reshape(N_PAD, 1)
    out = _combine(s0, s1, c0, c1)
    return out[:N]


# speedup vs baseline: 2.0075x; 1.1013x over previous
import jax
import jax.numpy as jnp
from jax import lax
from jax.experimental import pallas as pl
from jax.experimental.pallas import tpu as pltpu
from jax.experimental.pallas import tpu_sc as plsc

N = 100000
N_PAD = 100096
NODES_PER_TILE = N_PAD // 16
WB_CHUNK = 368
WB_COPIES = NODES_PER_TILE // WB_CHUNK

SUB = 128
CHR = 8
CH = SUB * CHR
NCHUNKS = 3125


def _linear_relu_body(x_ref, w_ref, b_ref, o_ref):
    acc = jnp.dot(x_ref[...], w_ref[...], preferred_element_type=jnp.float32)
    o_ref[...] = jnp.maximum(acc + b_ref[...], 0.0)


def _linear_relu(emb_packed, w_block, b_tile):
    rows = emb_packed.shape[0]
    blk = 1000
    grid = rows // blk
    return pl.pallas_call(
        _linear_relu_body,
        grid=(grid,),
        in_specs=[
            pl.BlockSpec((blk, 128), lambda i: (i, 0)),
            pl.BlockSpec((128, 128), lambda i: (0, 0)),
            pl.BlockSpec((1, 128), lambda i: (0, 0)),
        ],
        out_specs=pl.BlockSpec((blk, 128), lambda i: (i, 0)),
        out_shape=jax.ShapeDtypeStruct((rows, 128), jnp.float32),
    )(emb_packed, w_block, b_tile)


def _scatter_body(h_hbm, idx_hbm, sums_hbm, cnts_hbm,
                  rows_v, idx_v, ones_v, z1_v, sem_s, sem_c,
                  acc_sum, acc_cnt):
    num_edges = h_hbm.shape[0]
    idx_rows_per_sec = num_edges // SUB
    cid = lax.axis_index("c")
    sid = lax.axis_index("s")
    nch = (NCHUNKS + 15 - sid) // 16

    for i in range(SUB // 16):
        ones_v[pl.ds(i * 16, 16)] = jnp.full((16,), 1.0, jnp.float32)

    def zr(i, _):
        rows_v[i] = jnp.zeros((16,), jnp.float32)
        return 0
    lax.fori_loop(0, WB_CHUNK, zr, 0)

    def zc(i, _):
        z1_v[pl.ds(i * 16, 16)] = jnp.zeros((16,), jnp.float32)
        return 0
    lax.fori_loop(0, WB_CHUNK // 16, zc, 0)

    node0 = sid * NODES_PER_TILE
    for k in range(WB_COPIES):
        off = pl.multiple_of(node0 + k * WB_CHUNK, 8)
        pltpu.sync_copy(rows_v.at[pl.ds(0, WB_CHUNK)],
                        acc_sum.at[pl.ds(off, WB_CHUNK)])
        pltpu.sync_copy(z1_v, acc_cnt.at[pl.ds(off, WB_CHUNK)])
    plsc.subcore_barrier()

    def chunk(i, _):
        c = 16 * i + sid
        irow = pl.multiple_of(cid * idx_rows_per_sec + c * CHR, 8)
        erow = pl.multiple_of(c * CH, 8)
        pltpu.sync_copy(idx_hbm.at[pl.ds(irow, CHR)], idx_v)
        pltpu.sync_copy(h_hbm.at[pl.ds(erow, CH)], rows_v)
        cps = []
        for j in range(CHR):
            cps.append(pltpu.async_copy(rows_v.at[pl.ds(j * SUB, SUB)],
                                        acc_sum.at[idx_v.at[j]], sem_s,
                                        add=True))
            cps.append(pltpu.async_copy(ones_v, acc_cnt.at[idx_v.at[j]],
                                        sem_c, add=True))
        for cp in cps:
            cp.wait()
        return 0
    lax.fori_loop(0, nch, chunk, 0)
    plsc.subcore_barrier()

    out0 = cid * N_PAD + node0
    for k in range(WB_COPIES):
        off = pl.multiple_of(node0 + k * WB_CHUNK, 8)
        off_o = pl.multiple_of(out0 + k * WB_CHUNK, 8)
        pltpu.sync_copy(acc_sum.at[pl.ds(off, WB_CHUNK)],
                        rows_v.at[pl.ds(0, WB_CHUNK)])
        pltpu.sync_copy(rows_v.at[pl.ds(0, WB_CHUNK)],
                        sums_hbm.at[pl.ds(off_o, WB_CHUNK)])
        pltpu.sync_copy(acc_cnt.at[pl.ds(off, WB_CHUNK)], z1_v)
        pltpu.sync_copy(z1_v, cnts_hbm.at[pl.ds(off_o, WB_CHUNK)])


def _sc_scatter(h, idx2d):
    mesh = plsc.VectorSubcoreMesh(core_axis_name="c", subcore_axis_name="s",
                                  num_cores=2, num_subcores=16)
    f = pl.kernel(
        _scatter_body,
        out_type=[
            jax.ShapeDtypeStruct((2 * N_PAD, 16), jnp.float32),
            jax.ShapeDtypeStruct((2 * N_PAD,), jnp.float32),
        ],
        mesh=mesh,
        scratch_types=[
            pltpu.VMEM((CH, 16), jnp.float32),
            pltpu.VMEM((CHR, SUB), jnp.int32),
            pltpu.VMEM((SUB,), jnp.float32),
            pltpu.VMEM((WB_CHUNK,), jnp.float32),
            pltpu.SemaphoreType.DMA,
            pltpu.SemaphoreType.DMA,
            pltpu.VMEM_SHARED((N_PAD, 16), jnp.float32),
            pltpu.VMEM_SHARED((N_PAD,), jnp.float32),
        ],
        compiler_params=pltpu.CompilerParams(use_tc_tiling_on_sc=False),
    )
    return f(h, idx2d)


CMB_CH = 256
CMB_NCH = N_PAD // CMB_CH


def _combine_body(s_hbm, c_hbm, o_hbm, s0v, s1v, c0v, c1v):
    cid = lax.axis_index("c")
    sid = lax.axis_index("s")
    wid = sid * 2 + cid
    nch = (CMB_NCH + 31 - wid) // 32

    def chunk(i, _):
        c = 32 * i + wid
        off = pl.multiple_of(c * CMB_CH, 8)
        off1 = pl.multiple_of(N_PAD + c * CMB_CH, 8)
        pltpu.sync_copy(s_hbm.at[pl.ds(off, CMB_CH)], s0v)
        pltpu.sync_copy(s_hbm.at[pl.ds(off1, CMB_CH)], s1v)
        pltpu.sync_copy(c_hbm.at[pl.ds(off, CMB_CH)], c0v)
        pltpu.sync_copy(c_hbm.at[pl.ds(off1, CMB_CH)], c1v)

        def grp(g, _):
            rc0 = 1.0 / jnp.maximum(c0v[pl.ds(g * 16, 16)], 1.0)
            rc1 = 1.0 / jnp.maximum(c1v[pl.ds(g * 16, 16)], 1.0)
            for k in range(16):
                n = g * 16 + k
                acc = s0v[n] * rc0[k] + s1v[n] * rc1[k]
                s0v[n] = 0.5 * acc
            return 0
        lax.fori_loop(0, CMB_CH // 16, grp, 0)
        pltpu.sync_copy(s0v, o_hbm.at[pl.ds(off, CMB_CH)])
        return 0
    lax.fori_loop(0, nch, chunk, 0)


def _sc_combine(sums, cnts):
    mesh = plsc.VectorSubcoreMesh(core_axis_name="c", subcore_axis_name="s",
                                  num_cores=2, num_subcores=16)
    f = pl.kernel(
        _combine_body,
        out_type=[jax.ShapeDtypeStruct((N_PAD, 16), jnp.float32)],
        mesh=mesh,
        scratch_types=[
            pltpu.VMEM((CMB_CH, 16), jnp.float32),
            pltpu.VMEM((CMB_CH, 16), jnp.float32),
            pltpu.VMEM((CMB_CH,), jnp.float32),
            pltpu.VMEM((CMB_CH,), jnp.float32),
        ],
        compiler_params=pltpu.CompilerParams(use_tc_tiling_on_sc=False),
    )
    out = f(sums, cnts)
    return out[0] if isinstance(out, (list, tuple)) else out


def kernel(edge_emb, edge_index, num_nodes, W, b):
    num_edges = edge_emb.shape[0]
    emb_packed = edge_emb.reshape(num_edges // 8, 128)
    w_block = jnp.kron(jnp.eye(8, dtype=jnp.float32), W.T)
    b_tile = jnp.tile(b, 8).reshape(1, 128)
    h = _linear_relu(emb_packed, w_block, b_tile).reshape(num_edges, 16)

    idx2d = edge_index.reshape(2 * num_edges // SUB, SUB)
    sums, cnts = _sc_scatter(h, idx2d)

    out = _sc_combine(sums, cnts)
    return out[:N]
